# Initial kernel scaffold; baseline (speedup 1.0000x reference)
import jax
import jax.numpy as jnp
from jax.experimental import pallas as pl


def kernel(x, edge_index, batch, emb, W1, b1, W2, b2, W3, b3, Wf, bf):
    raise NotImplementedError("write your pallas kernel here")


# speedup vs baseline: 9.0698x; 9.0698x over previous
import functools

import jax
import jax.numpy as jnp
from jax import lax
from jax.experimental import pallas as pl
from jax.experimental.pallas import tpu as pltpu
from jax.experimental.pallas import tpu_sc as plsc

NC = 2
NS = 16
NW = NC * NS
D = 128
G = 100
GP = 104
EK = 80

@functools.cache
def _mesh():
    return plsc.VectorSubcoreMesh(core_axis_name="c", subcore_axis_name="s",
                                  num_cores=NC, num_subcores=NS)


def _prep_body(npad, echunks, x_hbm, dst_hbm, emb_hbm, z128_hbm, ones_hbm,
               deg_hbm, h0_hbm, idx_v, rows_v, ones_v, acc, sem):
    c = lax.axis_index("c")
    s = lax.axis_index("s")
    w = c * NS + s
    rpw = npad // NW
    rps = npad // NS

    for j in range(rpw // EK):
        base = w * rpw + j * EK
        pltpu.sync_copy(x_hbm.at[pl.ds(base, EK)], idx_v)
        pltpu.async_copy(emb_hbm.at[idx_v], rows_v, sem).wait()
        pltpu.sync_copy(rows_v, h0_hbm.at[pl.ds(base, EK)])

    pltpu.sync_copy(z128_hbm, acc.at[pl.ds(s * rps, rps)])
    pltpu.sync_copy(ones_hbm, ones_v)
    plsc.subcore_barrier()

    epw = echunks * EK

    def edge_step(j, _):
        base = w * epw + j * EK
        pltpu.sync_copy(dst_hbm.at[pl.ds(base, EK)], idx_v)
        pltpu.sync_copy(ones_v, acc.at[idx_v], add=True)
        return 0

    lax.fori_loop(0, echunks, edge_step, 0)
    plsc.subcore_barrier()
    pltpu.sync_copy(acc.at[pl.ds(s * rps, rps)],
                    deg_hbm.at[c, pl.ds(s * rps, rps)])


def _scatter_body(npad, echunks, g_hbm, src_hbm, dst_hbm, z128_hbm,
                  sh_hbm, srcv, dstv, rows_v, acc, sem):
    c = lax.axis_index("c")
    s = lax.axis_index("s")
    w = c * NS + s
    rps = npad // NS

    pltpu.sync_copy(z128_hbm, acc.at[pl.ds(s * rps, rps)])
    plsc.subcore_barrier()

    epw = echunks * EK

    def edge_step(j, _):
        base = w * epw + j * EK
        pltpu.sync_copy(src_hbm.at[pl.ds(base, EK)], srcv)
        pltpu.sync_copy(dst_hbm.at[pl.ds(base, EK)], dstv)
        pltpu.async_copy(g_hbm.at[srcv], rows_v, sem).wait()
        pltpu.sync_copy(rows_v, acc.at[dstv], add=True)
        return 0

    lax.fori_loop(0, echunks, edge_step, 0)
    plsc.subcore_barrier()
    pltpu.sync_copy(acc.at[pl.ds(s * rps, rps)],
                    sh_hbm.at[c, pl.ds(s * rps, rps)])


def _pool_body(npad, h3_hbm, batch_hbm, zpool_hbm, pool_hbm,
               buf, rows_v, bv):
    c = lax.axis_index("c")
    s = lax.axis_index("s")
    w = c * NS + s
    rpw = npad // NW

    pltpu.sync_copy(zpool_hbm, buf)
    base = w * rpw
    pltpu.sync_copy(h3_hbm.at[pl.ds(base * D, rpw * D)], rows_v)
    pltpu.sync_copy(batch_hbm.at[pl.ds(base, rpw)], bv)

    def row_blk(jb, _):
        bvec = bv[pl.ds(jb * 16, 16)]
        for ii in range(16):
            gb = bvec[ii] * D
            rb = (jb * 16 + ii) * D
            for k in range(D // 16):
                v = rows_v[pl.ds(rb + k * 16, 16)]
                cur = buf[pl.ds(gb + k * 16, 16)]
                buf[pl.ds(gb + k * 16, 16)] = jnp.maximum(cur, v)
        return 0

    lax.fori_loop(0, rpw // 16, row_blk, 0)
    pltpu.sync_copy(buf, pool_hbm.at[w])


@functools.cache
def _make_prep(npad, echunks):
    return pl.kernel(
        functools.partial(_prep_body, npad, echunks),
        out_type=(jax.ShapeDtypeStruct((NC, npad, D), jnp.float32),
                  jax.ShapeDtypeStruct((npad, D), jnp.float32)),
        mesh=_mesh(),
        scratch_types=[
            pltpu.VMEM((EK,), jnp.int32),
            pltpu.VMEM((EK, D), jnp.float32),
            pltpu.VMEM((EK, D), jnp.float32),
            pltpu.VMEM_SHARED((npad, D), jnp.float32),
            pltpu.SemaphoreType.DMA,
        ],
    )


@functools.cache
def _make_scatter(npad, echunks):
    return pl.kernel(
        functools.partial(_scatter_body, npad, echunks),
        out_type=jax.ShapeDtypeStruct((NC, npad, D), jnp.float32),
        mesh=_mesh(),
        scratch_types=[
            pltpu.VMEM((EK,), jnp.int32),
            pltpu.VMEM((EK,), jnp.int32),
            pltpu.VMEM((EK, D), jnp.float32),
            pltpu.VMEM_SHARED((npad, D), jnp.float32),
            pltpu.SemaphoreType.DMA,
        ],
    )


@functools.cache
def _make_pool(npad):
    rpw = npad // NW
    return pl.kernel(
        functools.partial(_pool_body, npad),
        out_type=jax.ShapeDtypeStruct((NW, GP * D), jnp.float32),
        mesh=_mesh(),
        scratch_types=[
            pltpu.VMEM((GP * D,), jnp.float32),
            pltpu.VMEM((rpw * D,), jnp.float32),
            pltpu.VMEM((rpw,), jnp.int32),
        ],
    )


BLK = 256


def _tc1_body(dega, degb, h0, W, dinv_o, g_o):
    dinv = lax.rsqrt(dega[:, :1] + degb[:, :1] + 1.0)
    dinv_o[...] = dinv
    g_o[...] = dinv * jnp.dot(h0[...], W[...], preferred_element_type=jnp.float32)


def _tc_mid_body(sa, sb, g, dinv, b, W, out):
    dv = dinv[...]
    h = jnp.maximum(dv * (sa[...] + sb[...] + g[...]) + b[...], 0.0)
    out[...] = dv * jnp.dot(h, W[...], preferred_element_type=jnp.float32)


def _tc_last_body(nreal, sa, sb, g, dinv, b, out):
    h = jnp.maximum(dinv[...] * (sa[...] + sb[...] + g[...]) + b[...], 0.0)
    ridx = pl.program_id(0) * BLK + lax.broadcasted_iota(jnp.int32, (BLK, D), 0)
    out[...] = jnp.where(ridx < nreal, h, 0.0)


def _tc_pool_body(pool, Wf, bf, out):
    pooled = jnp.max(pool[...], axis=0)
    out[...] = jnp.dot(pooled, Wf[...], preferred_element_type=jnp.float32) + bf[0, 0]


def _row_spec(width):
    return pl.BlockSpec((BLK, width), lambda i: (i, 0))


def _full_spec(shape):
    return pl.BlockSpec(shape, lambda i: tuple(0 for _ in shape))


@functools.cache
def _make_tc1(npad):
    return pl.pallas_call(
        _tc1_body,
        grid=(npad // BLK,),
        in_specs=[_row_spec(D), _row_spec(D), _row_spec(D), _full_spec((D, D))],
        out_specs=[_row_spec(1), _row_spec(D)],
        out_shape=(jax.ShapeDtypeStruct((npad, 1), jnp.float32),
                   jax.ShapeDtypeStruct((npad, D), jnp.float32)),
    )


@functools.cache
def _make_tc_mid(npad):
    return pl.pallas_call(
        _tc_mid_body,
        grid=(npad // BLK,),
        in_specs=[_row_spec(D), _row_spec(D), _row_spec(D), _row_spec(1),
                  _full_spec((1, D)), _full_spec((D, D))],
        out_specs=_row_spec(D),
        out_shape=jax.ShapeDtypeStruct((npad, D), jnp.float32),
    )


@functools.cache
def _make_tc_last(npad, nreal):
    return pl.pallas_call(
        functools.partial(_tc_last_body, nreal),
        grid=(npad // BLK,),
        in_specs=[_row_spec(D), _row_spec(D), _row_spec(D), _row_spec(1),
                  _full_spec((1, D))],
        out_specs=_row_spec(D),
        out_shape=jax.ShapeDtypeStruct((npad, D), jnp.float32),
    )


@functools.cache
def _make_tc_pool():
    return pl.pallas_call(
        _tc_pool_body,
        in_specs=[pl.BlockSpec((NW, GP, D), lambda: (0, 0, 0)),
                  pl.BlockSpec((D, 1), lambda: (0, 0)),
                  pl.BlockSpec((1, 1), lambda: (0, 0), memory_space=pltpu.SMEM)],
        out_specs=pl.BlockSpec((GP, 1), lambda: (0, 0)),
        out_shape=jax.ShapeDtypeStruct((GP, 1), jnp.float32),
    )


def kernel(x, edge_index, batch, emb, W1, b1, W2, b2, W3, b3, Wf, bf):
    n = x.shape[0]
    e = edge_index.shape[1]
    npad = ((n + NW * 16 - 1) // (NW * 16)) * (NW * 16)
    echunks = e // (NW * EK)

    x_p = jnp.concatenate([x.astype(jnp.int32),
                           jnp.zeros((npad - n,), jnp.int32)])
    batch_p = jnp.concatenate([batch.astype(jnp.int32),
                               jnp.zeros((npad - n,), jnp.int32)])
    src = edge_index[0].astype(jnp.int32)
    dst = edge_index[1].astype(jnp.int32)

    rps = npad // NS
    z128 = jnp.zeros((rps, D), jnp.float32)
    zpool = jnp.zeros((GP * D,), jnp.float32)
    ones128 = jnp.ones((EK, D), jnp.float32)

    deg2, h0 = _make_prep(npad, echunks)(x_p, dst, emb, z128, ones128)
    dinv, g1 = _make_tc1(npad)(deg2[0], deg2[1], h0, W1)

    b1r = b1.reshape(1, D)
    b2r = b2.reshape(1, D)
    b3r = b3.reshape(1, D)

    sh1 = _make_scatter(npad, echunks)(g1, src, dst, z128)
    g2 = _make_tc_mid(npad)(sh1[0], sh1[1], g1, dinv, b1r, W2)
    sh2 = _make_scatter(npad, echunks)(g2, src, dst, z128)
    g3 = _make_tc_mid(npad)(sh2[0], sh2[1], g2, dinv, b2r, W3)
    sh3 = _make_scatter(npad, echunks)(g3, src, dst, z128)
    h3 = _make_tc_last(npad, n)(sh3[0], sh3[1], g3, dinv, b3r)

    pool = _make_pool(npad)(h3.reshape(-1), batch_p, zpool)
    out = _make_tc_pool()(pool.reshape(NW, GP, D), Wf, bf.reshape(1, 1))
    return out[:G, 0]
